# baseline (speedup 1.0000x reference)
import functools

import jax
import jax.numpy as jnp
from jax import lax
from jax.experimental import pallas as pl
from jax.experimental.pallas import tpu as pltpu
from jax.experimental.pallas import tpu_sc as plsc

B, N, D, H, K = 64, 4096, 256, 256, 256
BN = B * N
CAP = 512
NV = CAP // 16
TM = 16384

import numpy as np

_I32_MIN = np.int32(-2147483648)
_SENTINEL = np.int32(2147483647)


def _scores_body(x_ref, w1_ref, w2_ref, b1_ref, s_ref):
    h = lax.dot_general(x_ref[...].astype(jnp.bfloat16),
                        w1_ref[...].astype(jnp.bfloat16),
                        (((1,), (1,)), ((), ())),
                        preferred_element_type=jnp.float32)
    h = jnp.maximum(h + b1_ref[...], 0.0)
    s = lax.dot_general(w2_ref[...], h, (((1,), (1,)), ((), ())),
                        preferred_element_type=jnp.float32)
    s_ref[...] = s.reshape(1, 1, TM)


def _scores(x2d, W1, W2, b1r):
    grid = BN // TM
    out = pl.pallas_call(
        _scores_body,
        grid=(grid,),
        in_specs=[
            pl.BlockSpec((TM, D), lambda i: (i, 0)),
            pl.BlockSpec((H, D), lambda i: (0, 0)),
            pl.BlockSpec((1, H), lambda i: (0, 0)),
            pl.BlockSpec((1, H), lambda i: (0, 0)),
        ],
        out_specs=pl.BlockSpec((1, 1, TM), lambda i: (i, 0, 0)),
        out_shape=jax.ShapeDtypeStruct((grid, 1, TM), jnp.float32),
    )(x2d, W1, W2, b1r)
    return out.reshape(B, N)


def _keys_body(s_ref, g_ref, b2_ref, u_ref):
    sb = s_ref[...] + b2_ref[0]
    m = jnp.max(sb, axis=1, keepdims=True)
    e = jnp.exp(sb - m)
    tot = jnp.sum(e, axis=1, keepdims=True)
    p = e / tot
    key = jnp.log(p + 1e-20) + g_ref[...]
    bi = lax.bitcast_convert_type(key, jnp.int32)
    u = jnp.where(bi < 0, bi ^ _I32_MIN, ~bi)
    u_ref[...] = u


def _keys(s2, gum, b2):
    return pl.pallas_call(
        _keys_body,
        in_specs=[
            pl.BlockSpec(memory_space=pltpu.VMEM),
            pl.BlockSpec(memory_space=pltpu.VMEM),
            pl.BlockSpec(memory_space=pltpu.SMEM),
        ],
        out_specs=pl.BlockSpec(memory_space=pltpu.VMEM),
        out_shape=jax.ShapeDtypeStruct((B, N), jnp.int32),
    )(s2, gum, b2)


def _iota16():
    return lax.iota(jnp.int32, 16)


def _ld(ref, base):
    return plsc.load_gather(ref, [base + _iota16()])


def _st(ref, base, x):
    plsc.store_scatter(ref, [base + _iota16()], x)


def _hist_level(shift, masked, prefix, base, u_row, hist, bsum):
    UNR = 8
    lanebase = _iota16() * 256

    def zero_body(i, _):
        for k in range(UNR):
            _st(hist, (i * UNR + k) * 16, jnp.zeros(16, jnp.int32))
        return 0

    lax.fori_loop(0, 256 // UNR, zero_body, 0)

    ones = jnp.ones(16, jnp.int32)

    def acc_body(v, _):
        for k in range(UNR):
            uu = _ld(u_row, (v * UNR + k) * 16)
            ub = uu ^ _I32_MIN
            bin_ = lax.shift_right_logical(ub, jnp.int32(shift)) & 255
            flat = lanebase + bin_
            if not masked:
                plsc.addupdate_scatter(hist, [flat], ones)
            else:
                pfx = lax.shift_right_logical(ub, jnp.int32(shift + 8))
                plsc.addupdate_scatter(hist, [flat], ones, mask=pfx == prefix)
        return 0

    lax.fori_loop(0, (N // 16) // UNR, acc_body, 0)

    for j in range(16):
        acc = _ld(hist, j * 16)
        for l in range(1, 16):
            acc = acc + _ld(hist, l * 256 + j * 16)
        _st(bsum, j * 16, acc)
    carry = jnp.int32(0)
    for j in range(16):
        c = plsc.cumsum(_ld(bsum, j * 16)) + carry
        _st(bsum, 256 + j * 16, c)
        carry = jnp.max(c)
    tbin = jnp.int32(255)
    found = jnp.int32(0)
    for j in range(16):
        c = _ld(bsum, 256 + j * 16)
        ffs = plsc.all_reduce_ffs(base + c >= K)
        f0 = jnp.max(ffs)
        hit = jnp.logical_and(found == 0, f0 < 16)
        tbin = jnp.where(hit, j * 16 + f0, tbin)
        found = jnp.where(hit, jnp.int32(1), found)
    tv = jnp.full((16,), tbin, jnp.int32)
    size = jnp.max(plsc.load_gather(bsum, [tv]))
    cumt = jnp.max(plsc.load_gather(bsum, [tv + 256]))
    return tbin, cumt - size, size


def _select_row(r, u_hbm, u_row, hist, bsum, lvl_out, cand_u, cand_i, k2v,
                gidx):
    pltpu.sync_copy(u_hbm.at[r], u_row)

    t1, cb1, _sz1 = _hist_level(24, False, jnp.int32(0), jnp.int32(0),
                                u_row, hist, bsum)
    prefix = t1
    base = cb1
    t2, cb2, sz2 = _hist_level(16, True, prefix, base, u_row, hist, bsum)
    prefix = lax.shift_left(prefix, jnp.int32(8)) | t2
    base = base + cb2
    shift = jnp.int32(16)
    size = sz2

    for extra_shift in (8, 0):
        need = size > CAP - K

        @pl.when(need)
        def _(extra_shift=extra_shift):
            t, cb, sz = _hist_level(extra_shift, True, prefix, base,
                                    u_row, hist, bsum)
            lvl_out[0] = t
            lvl_out[1] = cb
            lvl_out[2] = sz

        prefix = jnp.where(need,
                           lax.shift_left(prefix, jnp.int32(8)) | lvl_out[0],
                           prefix)
        base = jnp.where(need, base + lvl_out[1], base)
        size = jnp.where(need, lvl_out[2], size)
        shift = jnp.where(need, jnp.int32(extra_shift), shift)

    umax = (lax.shift_left(prefix + 1, shift) - 1) ^ _I32_MIN

    def init_body(i, _):
        _st(cand_u, i * 16, jnp.full(16, _SENTINEL, jnp.int32))
        _st(cand_i, i * 16, jnp.zeros(16, jnp.int32))
        return 0

    lax.fori_loop(0, NV, init_body, 0)

    def compact_body(v, off):
        for k in range(2):
            uu = _ld(u_row, (v * 2 + k) * 16)
            ii = (v * 2 + k) * 16 + _iota16()
            msk = uu <= umax
            cs = plsc.cumsum(msk.astype(jnp.int32))
            pos = off + cs - 1
            sel = jnp.logical_and(msk, pos < CAP)
            plsc.store_scatter(cand_u, [pos], uu, mask=sel)
            plsc.store_scatter(cand_i, [pos], ii, mask=sel)
            off = off + jnp.max(cs)
        return off

    cnt = lax.fori_loop(0, (N // 16) // 2, compact_body, jnp.int32(0))
    cv = jnp.minimum((cnt + 15) // 16, NV)

    def vsort_kv(v, _):
        k = _ld(cand_u, v * 16)
        val = _ld(cand_i, v * 16)
        sk, sv = plsc.sort_key_val(k, val)
        _st(cand_u, v * 16, sk)
        _st(cand_i, v * 16, sv)
        return 0

    lax.fori_loop(0, cv, vsort_kv, 0)

    def cross_kv(m):
        jlim = jnp.minimum((cv + 2 * m - 1) // (2 * m) * m, NV // 2)

        def body(j, _):
            blk = (j // m) * (2 * m)
            i = j % m
            av = (blk + i) * 16
            bv = (blk + 2 * m - 1 - i) * 16
            au = _ld(cand_u, av)
            ai = _ld(cand_i, av)
            bu0 = _ld(cand_u, bv)
            bi0 = _ld(cand_i, bv)
            bu = lax.rev(bu0, (0,))
            bival = lax.rev(bi0, (0,))
            c = au <= bu
            lo_u = jnp.where(c, au, bu)
            lo_i = jnp.where(c, ai, bival)
            hi_u = jnp.where(c, bu, au)
            hi_i = jnp.where(c, bival, ai)
            _st(cand_u, av, lo_u)
            _st(cand_i, av, lo_i)
            _st(cand_u, bv, lax.rev(hi_u, (0,)))
            _st(cand_i, bv, lax.rev(hi_i, (0,)))
            return 0

        lax.fori_loop(0, jlim, body, 0)

    def stage_kv(d):
        tlim = jnp.minimum((cv // (2 * d)) * d + jnp.minimum(cv % (2 * d), d),
                           NV // 2)

        def body(t, _):
            v1 = ((t // d) * (2 * d) + t % d) * 16
            v2 = v1 + d * 16
            au = _ld(cand_u, v1)
            ai = _ld(cand_i, v1)
            bu = _ld(cand_u, v2)
            bival = _ld(cand_i, v2)
            c = au <= bu
            _st(cand_u, v1, jnp.where(c, au, bu))
            _st(cand_i, v1, jnp.where(c, ai, bival))
            _st(cand_u, v2, jnp.where(c, bu, au))
            _st(cand_i, v2, jnp.where(c, bival, ai))
            return 0

        lax.fori_loop(0, tlim, body, 0)

    m = 1
    while m < NV:
        cross_kv(m)
        d = m // 2
        while d >= 1:
            stage_kv(d)
            d //= 2
        lax.fori_loop(0, cv, vsort_kv, 0)
        m *= 2

    def g_body(v, carry):
        carry_max, ties = carry
        uu = _ld(cand_u, v * 16)
        pidx = jnp.maximum(v * 16 - 1 + _iota16(), 0)
        prev = plsc.load_gather(cand_u, [pidx])
        neq = uu != prev
        neq = jnp.logical_or(neq, (v * 16 + _iota16()) == 0)
        tie = jnp.logical_and(jnp.logical_not(neq), uu != _SENTINEL)
        start = jnp.where(neq, v * 16 + _iota16(), 0)
        cm = jnp.maximum(plsc.cummax(start), carry_max)
        k2 = cm * 4096 + _ld(cand_i, v * 16)
        _st(k2v, v * 16, k2)
        return jnp.max(cm), ties + jnp.sum(tie.astype(jnp.int32))

    _, n_ties = lax.fori_loop(0, NV, g_body, (jnp.int32(0), jnp.int32(0)))

    nv2 = jnp.where(n_ties > 0, jnp.int32(NV), jnp.int32(0))
    half2 = jnp.where(n_ties > 0, jnp.int32(NV // 2), jnp.int32(0))

    def vsort_k(v, _):
        k = _ld(k2v, v * 16)
        (sk,) = lax.sort((k,), dimension=0)
        _st(k2v, v * 16, sk)
        return 0

    lax.fori_loop(0, nv2, vsort_k, 0)

    def cross_k(m):
        def body(j, _):
            blk = (j // m) * (2 * m)
            i = j % m
            av = (blk + i) * 16
            bv = (blk + 2 * m - 1 - i) * 16
            a = _ld(k2v, av)
            b_ = lax.rev(_ld(k2v, bv), (0,))
            _st(k2v, av, jnp.minimum(a, b_))
            _st(k2v, bv, lax.rev(jnp.maximum(a, b_), (0,)))
            return 0

        lax.fori_loop(0, half2, body, 0)

    def stage_k(d):
        def body(t, _):
            v1 = ((t // d) * (2 * d) + t % d) * 16
            v2 = v1 + d * 16
            a = _ld(k2v, v1)
            b_ = _ld(k2v, v2)
            _st(k2v, v1, jnp.minimum(a, b_))
            _st(k2v, v2, jnp.maximum(a, b_))
            return 0

        lax.fori_loop(0, half2, body, 0)

    m = 1
    while m < NV:
        cross_k(m)
        d = m // 2
        while d >= 1:
            stage_k(d)
            d //= 2
        lax.fori_loop(0, nv2, vsort_k, 0)
        m *= 2

    def gidx_body(i, _):
        kk = _ld(k2v, i * 16)
        _st(gidx, i * 16, (kk & 4095) + r * N)
        return 0

    lax.fori_loop(0, K // 16, gidx_body, 0)


def _sc_select_gather(u, x2d):
    info = plsc.get_sparse_core_info()
    nc, ns = info.num_cores, info.num_subcores
    nw = nc * ns
    rows_per_w = B // nw
    mesh = plsc.VectorSubcoreMesh(core_axis_name="c", subcore_axis_name="s")

    @functools.partial(
        pl.kernel,
        out_type=jax.ShapeDtypeStruct((B * K, D), jnp.float32),
        mesh=mesh,
        scratch_types=[
            pltpu.VMEM((N,), jnp.int32),
            pltpu.VMEM((256 * 16,), jnp.int32),
            pltpu.VMEM((512,), jnp.int32),
            pltpu.SMEM((4,), jnp.int32),
            pltpu.VMEM((CAP,), jnp.int32),
            pltpu.VMEM((CAP,), jnp.int32),
            pltpu.VMEM((CAP,), jnp.int32),
            pltpu.VMEM((K,), jnp.int32),
            pltpu.VMEM((K,), jnp.int32),
            pltpu.VMEM((K, D), jnp.float32),
            pltpu.SemaphoreType.DMA,
            pltpu.SemaphoreType.DMA,
        ],
        compiler_params=pltpu.CompilerParams(needs_layout_passes=False),
    )
    def sc_kernel(u_hbm, x_hbm, out_hbm, u_row, hist, bsum, lvl_out, cand_u,
                  cand_i, k2v, gidx0, gidx1, rows, sem, sem2):
        wid = lax.axis_index("s") * nc + lax.axis_index("c")

        def row_body(t, _):
            r = wid * rows_per_w + t
            _select_row(r, u_hbm, u_row, hist, bsum, lvl_out, cand_u, cand_i,
                        k2v, gidx0)
            pltpu.async_copy(x_hbm.at[gidx0], rows, sem).wait()
            pltpu.sync_copy(rows, out_hbm.at[pl.ds(r * K, K)])
            return 0

        lax.fori_loop(0, rows_per_w, row_body, 0)

    return sc_kernel(u, x2d)


def _hsel_body(x_ref, w1_ref, b1_ref, o_ref):
    h = lax.dot_general(x_ref[...].astype(jnp.bfloat16),
                        w1_ref[...].astype(jnp.bfloat16),
                        (((1,), (1,)), ((), ())),
                        preferred_element_type=jnp.float32)
    o_ref[...] = jnp.maximum(h + b1_ref[...], 0.0)


def _hsel(xsel, W1, b1r):
    grid = (B * K) // TM
    return pl.pallas_call(
        _hsel_body,
        grid=(grid,),
        in_specs=[
            pl.BlockSpec((TM, D), lambda i: (i, 0)),
            pl.BlockSpec((H, D), lambda i: (0, 0)),
            pl.BlockSpec((1, H), lambda i: (0, 0)),
        ],
        out_specs=pl.BlockSpec((TM, H), lambda i: (i, 0)),
        out_shape=jax.ShapeDtypeStruct((B * K, H), jnp.float32),
    )(xsel, W1, b1r)


def kernel(x, W1, b1, W2, b2):
    x2d = x.reshape(BN, D)
    b1r = b1.reshape(1, H)
    gkey = jax.random.key(42)
    u01 = jax.random.uniform(gkey, (B, N), jnp.float32,
                             minval=1e-7, maxval=1.0 - 1e-7)
    gum = -jnp.log(-jnp.log(u01))

    s2 = _scores(x2d, W1, W2, b1r)
    u = _keys(s2, gum, b2)
    xsel = _sc_select_gather(u, x2d)
    hsel = _hsel(xsel, W1, b1r)
    return hsel.reshape(B, K, D)


# speedup vs baseline: 1.0020x; 1.0020x over previous
import functools

import jax
import jax.numpy as jnp
from jax import lax
from jax.experimental import pallas as pl
from jax.experimental.pallas import tpu as pltpu
from jax.experimental.pallas import tpu_sc as plsc

B, N, D, H, K = 64, 4096, 256, 256, 256
BN = B * N
CAP = 512
NV = CAP // 16
TM = 16384

import numpy as np

_I32_MIN = np.int32(-2147483648)
_SENTINEL = np.int32(2147483647)


def _scores_body(x_ref, w1_ref, w2_ref, b1_ref, s_ref):
    h = lax.dot_general(x_ref[...].astype(jnp.bfloat16),
                        w1_ref[...].astype(jnp.bfloat16),
                        (((1,), (1,)), ((), ())),
                        preferred_element_type=jnp.float32)
    h = jnp.maximum(h + b1_ref[...], 0.0)
    s = lax.dot_general(w2_ref[...], h, (((1,), (1,)), ((), ())),
                        preferred_element_type=jnp.float32)
    s_ref[...] = s.reshape(1, 1, TM)


def _scores(x2d, W1, W2, b1r):
    grid = BN // TM
    out = pl.pallas_call(
        _scores_body,
        grid=(grid,),
        in_specs=[
            pl.BlockSpec((TM, D), lambda i: (i, 0)),
            pl.BlockSpec((H, D), lambda i: (0, 0)),
            pl.BlockSpec((1, H), lambda i: (0, 0)),
            pl.BlockSpec((1, H), lambda i: (0, 0)),
        ],
        out_specs=pl.BlockSpec((1, 1, TM), lambda i: (i, 0, 0)),
        out_shape=jax.ShapeDtypeStruct((grid, 1, TM), jnp.float32),
    )(x2d, W1, W2, b1r)
    return out.reshape(B, N)


def _keys_body(s_ref, g_ref, b2_ref, u_ref):
    sb = s_ref[...] + b2_ref[0]
    m = jnp.max(sb, axis=1, keepdims=True)
    e = jnp.exp(sb - m)
    tot = jnp.sum(e, axis=1, keepdims=True)
    p = e / tot
    key = jnp.log(p + 1e-20) + g_ref[...]
    bi = lax.bitcast_convert_type(key, jnp.int32)
    u = jnp.where(bi < 0, bi ^ _I32_MIN, ~bi)
    u_ref[...] = u


def _keys(s2, gum, b2):
    return pl.pallas_call(
        _keys_body,
        in_specs=[
            pl.BlockSpec(memory_space=pltpu.VMEM),
            pl.BlockSpec(memory_space=pltpu.VMEM),
            pl.BlockSpec(memory_space=pltpu.SMEM),
        ],
        out_specs=pl.BlockSpec(memory_space=pltpu.VMEM),
        out_shape=jax.ShapeDtypeStruct((B, N), jnp.int32),
    )(s2, gum, b2)


def _iota16():
    return lax.iota(jnp.int32, 16)


def _ld(ref, base):
    return plsc.load_gather(ref, [base + _iota16()])


def _st(ref, base, x):
    plsc.store_scatter(ref, [base + _iota16()], x)


def _hist_level(shift, masked, prefix, base, u_row, hist, bsum):
    UNR = 4
    lanebase = _iota16() * 256

    def zero_body(i, _):
        for k in range(UNR):
            _st(hist, (i * UNR + k) * 16, jnp.zeros(16, jnp.int32))
        return 0

    lax.fori_loop(0, 256 // UNR, zero_body, 0)

    ones = jnp.ones(16, jnp.int32)

    def acc_body(v, _):
        for k in range(UNR):
            uu = _ld(u_row, (v * UNR + k) * 16)
            ub = uu ^ _I32_MIN
            bin_ = lax.shift_right_logical(ub, jnp.int32(shift)) & 255
            flat = lanebase + bin_
            if not masked:
                plsc.addupdate_scatter(hist, [flat], ones)
            else:
                pfx = lax.shift_right_logical(ub, jnp.int32(shift + 8))
                plsc.addupdate_scatter(hist, [flat], ones, mask=pfx == prefix)
        return 0

    lax.fori_loop(0, (N // 16) // UNR, acc_body, 0)

    for j in range(16):
        acc = _ld(hist, j * 16)
        for l in range(1, 16):
            acc = acc + _ld(hist, l * 256 + j * 16)
        _st(bsum, j * 16, acc)
    carry = jnp.int32(0)
    for j in range(16):
        c = plsc.cumsum(_ld(bsum, j * 16)) + carry
        _st(bsum, 256 + j * 16, c)
        carry = jnp.max(c)
    tbin = jnp.int32(255)
    found = jnp.int32(0)
    for j in range(16):
        c = _ld(bsum, 256 + j * 16)
        ffs = plsc.all_reduce_ffs(base + c >= K)
        f0 = jnp.max(ffs)
        hit = jnp.logical_and(found == 0, f0 < 16)
        tbin = jnp.where(hit, j * 16 + f0, tbin)
        found = jnp.where(hit, jnp.int32(1), found)
    tv = jnp.full((16,), tbin, jnp.int32)
    size = jnp.max(plsc.load_gather(bsum, [tv]))
    cumt = jnp.max(plsc.load_gather(bsum, [tv + 256]))
    return tbin, cumt - size, size


def _select_row(r, u_hbm, u_row, hist, bsum, lvl_out, cand_u, cand_i, k2v,
                gidx):
    pltpu.sync_copy(u_hbm.at[r], u_row)

    t1, cb1, _sz1 = _hist_level(24, False, jnp.int32(0), jnp.int32(0),
                                u_row, hist, bsum)
    prefix = t1
    base = cb1
    t2, cb2, sz2 = _hist_level(16, True, prefix, base, u_row, hist, bsum)
    prefix = lax.shift_left(prefix, jnp.int32(8)) | t2
    base = base + cb2
    shift = jnp.int32(16)
    size = sz2

    for extra_shift in (8, 0):
        need = size > CAP - K

        @pl.when(need)
        def _(extra_shift=extra_shift):
            t, cb, sz = _hist_level(extra_shift, True, prefix, base,
                                    u_row, hist, bsum)
            lvl_out[0] = t
            lvl_out[1] = cb
            lvl_out[2] = sz

        prefix = jnp.where(need,
                           lax.shift_left(prefix, jnp.int32(8)) | lvl_out[0],
                           prefix)
        base = jnp.where(need, base + lvl_out[1], base)
        size = jnp.where(need, lvl_out[2], size)
        shift = jnp.where(need, jnp.int32(extra_shift), shift)

    umax = (lax.shift_left(prefix + 1, shift) - 1) ^ _I32_MIN

    def init_body(i, _):
        _st(cand_u, i * 16, jnp.full(16, _SENTINEL, jnp.int32))
        _st(cand_i, i * 16, jnp.zeros(16, jnp.int32))
        return 0

    lax.fori_loop(0, NV, init_body, 0)

    def compact_body(v, off):
        for k in range(2):
            uu = _ld(u_row, (v * 2 + k) * 16)
            ii = (v * 2 + k) * 16 + _iota16()
            msk = uu <= umax
            cs = plsc.cumsum(msk.astype(jnp.int32))
            pos = off + cs - 1
            sel = jnp.logical_and(msk, pos < CAP)
            plsc.store_scatter(cand_u, [pos], uu, mask=sel)
            plsc.store_scatter(cand_i, [pos], ii, mask=sel)
            off = off + jnp.max(cs)
        return off

    cnt = lax.fori_loop(0, (N // 16) // 2, compact_body, jnp.int32(0))
    cv = jnp.minimum((cnt + 15) // 16, NV)

    def vsort_kv(v, _):
        k = _ld(cand_u, v * 16)
        val = _ld(cand_i, v * 16)
        sk, sv = plsc.sort_key_val(k, val)
        _st(cand_u, v * 16, sk)
        _st(cand_i, v * 16, sv)
        return 0

    lax.fori_loop(0, cv, vsort_kv, 0)

    def cross_kv(m):
        jlim = jnp.minimum((cv + 2 * m - 1) // (2 * m) * m, NV // 2)

        def body(j, _):
            blk = (j // m) * (2 * m)
            i = j % m
            av = (blk + i) * 16
            bv = (blk + 2 * m - 1 - i) * 16
            au = _ld(cand_u, av)
            ai = _ld(cand_i, av)
            bu0 = _ld(cand_u, bv)
            bi0 = _ld(cand_i, bv)
            bu = lax.rev(bu0, (0,))
            bival = lax.rev(bi0, (0,))
            c = au <= bu
            lo_u = jnp.where(c, au, bu)
            lo_i = jnp.where(c, ai, bival)
            hi_u = jnp.where(c, bu, au)
            hi_i = jnp.where(c, bival, ai)
            _st(cand_u, av, lo_u)
            _st(cand_i, av, lo_i)
            _st(cand_u, bv, lax.rev(hi_u, (0,)))
            _st(cand_i, bv, lax.rev(hi_i, (0,)))
            return 0

        lax.fori_loop(0, jlim, body, 0)

    def stage_kv(d):
        tlim = jnp.minimum((cv // (2 * d)) * d + jnp.minimum(cv % (2 * d), d),
                           NV // 2)

        def body(t, _):
            v1 = ((t // d) * (2 * d) + t % d) * 16
            v2 = v1 + d * 16
            au = _ld(cand_u, v1)
            ai = _ld(cand_i, v1)
            bu = _ld(cand_u, v2)
            bival = _ld(cand_i, v2)
            c = au <= bu
            _st(cand_u, v1, jnp.where(c, au, bu))
            _st(cand_i, v1, jnp.where(c, ai, bival))
            _st(cand_u, v2, jnp.where(c, bu, au))
            _st(cand_i, v2, jnp.where(c, bival, ai))
            return 0

        lax.fori_loop(0, tlim, body, 0)

    m = 1
    while m < NV:
        cross_kv(m)
        d = m // 2
        while d >= 1:
            stage_kv(d)
            d //= 2
        lax.fori_loop(0, cv, vsort_kv, 0)
        m *= 2

    def g_body(v, carry):
        carry_max, ties = carry
        uu = _ld(cand_u, v * 16)
        pidx = jnp.maximum(v * 16 - 1 + _iota16(), 0)
        prev = plsc.load_gather(cand_u, [pidx])
        neq = uu != prev
        neq = jnp.logical_or(neq, (v * 16 + _iota16()) == 0)
        tie = jnp.logical_and(jnp.logical_not(neq), uu != _SENTINEL)
        start = jnp.where(neq, v * 16 + _iota16(), 0)
        cm = jnp.maximum(plsc.cummax(start), carry_max)
        k2 = cm * 4096 + _ld(cand_i, v * 16)
        _st(k2v, v * 16, k2)
        return jnp.max(cm), ties + jnp.sum(tie.astype(jnp.int32))

    _, n_ties = lax.fori_loop(0, NV, g_body, (jnp.int32(0), jnp.int32(0)))

    nv2 = jnp.where(n_ties > 0, jnp.int32(NV), jnp.int32(0))
    half2 = jnp.where(n_ties > 0, jnp.int32(NV // 2), jnp.int32(0))

    def vsort_k(v, _):
        k = _ld(k2v, v * 16)
        (sk,) = lax.sort((k,), dimension=0)
        _st(k2v, v * 16, sk)
        return 0

    lax.fori_loop(0, nv2, vsort_k, 0)

    def cross_k(m):
        def body(j, _):
            blk = (j // m) * (2 * m)
            i = j % m
            av = (blk + i) * 16
            bv = (blk + 2 * m - 1 - i) * 16
            a = _ld(k2v, av)
            b_ = lax.rev(_ld(k2v, bv), (0,))
            _st(k2v, av, jnp.minimum(a, b_))
            _st(k2v, bv, lax.rev(jnp.maximum(a, b_), (0,)))
            return 0

        lax.fori_loop(0, half2, body, 0)

    def stage_k(d):
        def body(t, _):
            v1 = ((t // d) * (2 * d) + t % d) * 16
            v2 = v1 + d * 16
            a = _ld(k2v, v1)
            b_ = _ld(k2v, v2)
            _st(k2v, v1, jnp.minimum(a, b_))
            _st(k2v, v2, jnp.maximum(a, b_))
            return 0

        lax.fori_loop(0, half2, body, 0)

    m = 1
    while m < NV:
        cross_k(m)
        d = m // 2
        while d >= 1:
            stage_k(d)
            d //= 2
        lax.fori_loop(0, nv2, vsort_k, 0)
        m *= 2

    def gidx_body(i, _):
        kk = _ld(k2v, i * 16)
        _st(gidx, i * 16, (kk & 4095) + r * N)
        return 0

    lax.fori_loop(0, K // 16, gidx_body, 0)


def _sc_select_gather(u, x2d):
    info = plsc.get_sparse_core_info()
    nc, ns = info.num_cores, info.num_subcores
    nw = nc * ns
    rows_per_w = B // nw
    mesh = plsc.VectorSubcoreMesh(core_axis_name="c", subcore_axis_name="s")

    @functools.partial(
        pl.kernel,
        out_type=jax.ShapeDtypeStruct((B * K, D), jnp.float32),
        mesh=mesh,
        scratch_types=[
            pltpu.VMEM((N,), jnp.int32),
            pltpu.VMEM((256 * 16,), jnp.int32),
            pltpu.VMEM((512,), jnp.int32),
            pltpu.SMEM((4,), jnp.int32),
            pltpu.VMEM((CAP,), jnp.int32),
            pltpu.VMEM((CAP,), jnp.int32),
            pltpu.VMEM((CAP,), jnp.int32),
            pltpu.VMEM((K,), jnp.int32),
            pltpu.VMEM((K,), jnp.int32),
            pltpu.VMEM((K, D), jnp.float32),
            pltpu.SemaphoreType.DMA,
            pltpu.SemaphoreType.DMA,
        ],
        compiler_params=pltpu.CompilerParams(needs_layout_passes=False),
    )
    def sc_kernel(u_hbm, x_hbm, out_hbm, u_row, hist, bsum, lvl_out, cand_u,
                  cand_i, k2v, gidx0, gidx1, rows, sem, sem2):
        wid = lax.axis_index("s") * nc + lax.axis_index("c")

        def row_body(t, _):
            r = wid * rows_per_w + t
            _select_row(r, u_hbm, u_row, hist, bsum, lvl_out, cand_u, cand_i,
                        k2v, gidx0)
            pltpu.async_copy(x_hbm.at[gidx0], rows, sem).wait()
            pltpu.sync_copy(rows, out_hbm.at[pl.ds(r * K, K)])
            return 0

        lax.fori_loop(0, rows_per_w, row_body, 0)

    return sc_kernel(u, x2d)


def _hsel_body(x_ref, w1_ref, b1_ref, o_ref):
    h = lax.dot_general(x_ref[...].astype(jnp.bfloat16),
                        w1_ref[...].astype(jnp.bfloat16),
                        (((1,), (1,)), ((), ())),
                        preferred_element_type=jnp.float32)
    o_ref[...] = jnp.maximum(h + b1_ref[...], 0.0)


def _hsel(xsel, W1, b1r):
    grid = (B * K) // TM
    return pl.pallas_call(
        _hsel_body,
        grid=(grid,),
        in_specs=[
            pl.BlockSpec((TM, D), lambda i: (i, 0)),
            pl.BlockSpec((H, D), lambda i: (0, 0)),
            pl.BlockSpec((1, H), lambda i: (0, 0)),
        ],
        out_specs=pl.BlockSpec((TM, H), lambda i: (i, 0)),
        out_shape=jax.ShapeDtypeStruct((B * K, H), jnp.float32),
    )(xsel, W1, b1r)


def kernel(x, W1, b1, W2, b2):
    x2d = x.reshape(BN, D)
    b1r = b1.reshape(1, H)
    gkey = jax.random.key(42)
    u01 = jax.random.uniform(gkey, (B, N), jnp.float32,
                             minval=1e-7, maxval=1.0 - 1e-7)
    gum = -jnp.log(-jnp.log(u01))

    s2 = _scores(x2d, W1, W2, b1r)
    u = _keys(s2, gum, b2)
    xsel = _sc_select_gather(u, x2d)
    hsel = _hsel(xsel, W1, b1r)
    return hsel.reshape(B, K, D)
